# Initial kernel scaffold; baseline (speedup 1.0000x reference)
import jax
import jax.numpy as jnp
from jax.experimental import pallas as pl


def kernel(pred, I):
    raise NotImplementedError("write your pallas kernel here")


# speedup vs baseline: 4.2256x; 4.2256x over previous
import jax
import jax.numpy as jnp
from jax.experimental import pallas as pl

KSIZE = 3
NSIZE = 7
SCALE = 4
H = 224
HS = H // SCALE
C = 3
C1 = C * KSIZE * KSIZE

_HIGH = jax.lax.Precision.HIGHEST


def _census_resize_body(xpad_ref, w_ref, ct_ref, raw_ref):
    x = xpad_ref[0]
    w = w_ref[...]
    center = x[:, 1:1 + H, 1:1 + H]
    acc = jnp.zeros((C, H, H), jnp.float32)
    for i in range(KSIZE):
        for j in range(KSIZE):
            acc = acc + jnp.tanh(x[:, i:i + H, j:j + H] - center)
    ct = acc * (1.0 / (KSIZE * KSIZE))

    def resize(m):
        t1 = jax.lax.dot_general(m, w, (((1,), (1,)), ((), ())),
                                 precision=_HIGH)
        t2 = jax.lax.dot_general(t1, w, (((1,), (1,)), ((), ())),
                                 precision=_HIGH)
        return t2

    ct_ref[0] = resize(ct)
    raw_ref[0] = resize(center)


def _match_body(pct_ref, praw_ref, nct_ref, nraw_ref, out_ref):
    pct = pct_ref[0]
    praw = praw_ref[0]
    best_d = None
    best_raw = None
    for img in range(2):
        nct = nct_ref[0, img]
        nraw = nraw_ref[0, img]
        for dy in range(NSIZE):
            for dx in range(NSIZE):
                dc = pct - nct[:, dy:dy + HS, dx:dx + HS]
                d = jnp.sum(dc * dc, axis=0)
                rr = praw - nraw[:, dy:dy + HS, dx:dx + HS]
                r = jnp.sum(rr * rr, axis=0)
                if best_d is None:
                    best_d, best_raw = d, r
                else:
                    upd = d < best_d
                    best_d = jnp.where(upd, d, best_d)
                    best_raw = jnp.where(upd, r, best_raw)
    s = jnp.sum(best_raw)
    out_ref[0] = jnp.full((8, 128), s, jnp.float32)


def _patches3(x):
    n, c, h, w = x.shape
    xp = jnp.pad(x, ((0, 0), (0, 0), (1, 1), (1, 1)), mode='reflect')
    cols = [xp[:, :, i:i + h, j:j + w]
            for i in range(KSIZE) for j in range(KSIZE)]
    p = jnp.stack(cols, axis=2)
    return p.reshape(n, c * KSIZE * KSIZE, h, w)


def kernel(pred, I):
    b = pred.shape[0]
    nmaps = 3 * b
    maps = jnp.concatenate([pred[:, None], I], axis=1)
    flat = maps.reshape(nmaps, C, H, H)
    xpad = jnp.pad(flat, ((0, 0), (0, 0), (1, 1), (1, 1)), mode='reflect')

    w_op = jax.image.resize(jnp.eye(H, dtype=jnp.float32), (HS, H),
                            method='cubic')

    r_ct, r_raw = pl.pallas_call(
        _census_resize_body,
        grid=(nmaps,),
        in_specs=[
            pl.BlockSpec((1, C, H + 2, H + 2), lambda m: (m, 0, 0, 0)),
            pl.BlockSpec((HS, H), lambda m: (0, 0)),
        ],
        out_specs=[
            pl.BlockSpec((1, C, HS, HS), lambda m: (m, 0, 0, 0)),
            pl.BlockSpec((1, C, HS, HS), lambda m: (m, 0, 0, 0)),
        ],
        out_shape=[
            jax.ShapeDtypeStruct((nmaps, C, HS, HS), jnp.float32),
            jax.ShapeDtypeStruct((nmaps, C, HS, HS), jnp.float32),
        ],
    )(xpad, w_op)

    r_ct = r_ct.reshape(b, 3, C, HS, HS)
    r_raw = r_raw.reshape(b, 3, C, HS, HS)

    p_ct = _patches3(r_ct[:, 0])
    p_raw = _patches3(r_raw[:, 0])
    n_ct = _patches3(r_ct[:, 1:].reshape(2 * b, C, HS, HS))
    n_raw = _patches3(r_raw[:, 1:].reshape(2 * b, C, HS, HS))
    pad2 = NSIZE // 2
    n_ct = jnp.pad(n_ct.reshape(b, 2, C1, HS, HS),
                   ((0, 0), (0, 0), (0, 0), (pad2, pad2), (pad2, pad2)),
                   mode='reflect')
    n_raw = jnp.pad(n_raw.reshape(b, 2, C1, HS, HS),
                    ((0, 0), (0, 0), (0, 0), (pad2, pad2), (pad2, pad2)),
                    mode='reflect')

    hp = HS + 2 * pad2
    partial = pl.pallas_call(
        _match_body,
        grid=(b,),
        in_specs=[
            pl.BlockSpec((1, C1, HS, HS), lambda i: (i, 0, 0, 0)),
            pl.BlockSpec((1, C1, HS, HS), lambda i: (i, 0, 0, 0)),
            pl.BlockSpec((1, 2, C1, hp, hp), lambda i: (i, 0, 0, 0, 0)),
            pl.BlockSpec((1, 2, C1, hp, hp), lambda i: (i, 0, 0, 0, 0)),
        ],
        out_specs=pl.BlockSpec((1, 8, 128), lambda i: (i, 0, 0)),
        out_shape=jax.ShapeDtypeStruct((b, 8, 128), jnp.float32),
    )(p_ct, p_raw, n_ct, n_raw)

    total = jnp.sum(partial[:, 0, 0])
    return total * (0.5 / (b * HS * HS * C1))


# speedup vs baseline: 6.1492x; 1.4552x over previous
import numpy as np
import jax
import jax.numpy as jnp
from jax.experimental import pallas as pl

KSIZE = 3
NSIZE = 7
H = 224
HS = 56
C = 3
C1 = C * KSIZE * KSIZE
PAD2 = NSIZE // 2

_HIGH = jax.lax.Precision.HIGHEST


def _resize_matrix(in_size, out_size):
    scale = out_size / in_size
    inv_scale = 1.0 / scale
    kernel_scale = max(inv_scale, 1.0)
    sample_f = (np.arange(out_size, dtype=np.float32) + 0.5) * inv_scale - 0.5
    x = np.abs(sample_f[None, :]
               - np.arange(in_size, dtype=np.float32)[:, None]) / kernel_scale
    x = x.astype(np.float32)
    w = (((1.5 * x - 2.5) * x * x + 1.0) * (x <= 1.0)
         + ((((-0.5 * x + 2.5) * x - 4.0) * x + 2.0)
            * ((x > 1.0) & (x < 2.0)))).astype(np.float32)
    total = w.sum(axis=0, keepdims=True)
    w = np.where(np.abs(total) > 1000 * np.finfo(np.float32).eps,
                 w / np.where(total != 0, total, 1), 0)
    w = np.where(((sample_f >= -0.5) & (sample_f <= in_size - 0.5))[None, :],
                 w, 0)
    return np.ascontiguousarray(w.T.astype(np.float32))


_RESIZE_W = _resize_matrix(H, HS)


def _rpad(x, p, axis):
    n = x.shape[axis]

    def sl(a, b):
        return tuple(slice(a, b) if d == axis else slice(None)
                     for d in range(x.ndim))

    parts = ([x[sl(t, t + 1)] for t in range(p, 0, -1)]
             + [x]
             + [x[sl(n - 1 - t, n - t)] for t in range(1, p + 1)])
    return jnp.concatenate(parts, axis=axis)


def _fused_body(pred_ref, i_ref, w_ref, out_ref):
    w = w_ref[...]

    def census(x):
        xp = _rpad(_rpad(x, 1, 1), 1, 2)
        acc = jnp.zeros((C, H, H), jnp.float32)
        for i in range(KSIZE):
            for j in range(KSIZE):
                acc = acc + jnp.tanh(xp[:, i:i + H, j:j + H] - x)
        return acc * (1.0 / (KSIZE * KSIZE))

    def resize(m):
        t1 = jax.lax.dot_general(m, w, (((1,), (1,)), ((), ())),
                                 precision=_HIGH)
        return jax.lax.dot_general(t1, w, (((1,), (1,)), ((), ())),
                                   precision=_HIGH)

    def patches(r):
        rp = _rpad(_rpad(r, 1, 1), 1, 2)
        cols = [rp[:, i:i + HS, j:j + HS]
                for i in range(KSIZE) for j in range(KSIZE)]
        return jnp.stack(cols, axis=1).reshape(C1, HS, HS)

    maps = [pred_ref[0], i_ref[0, 0], i_ref[0, 1]]
    rz_ct = [resize(census(m)) for m in maps]
    rz_raw = [resize(m) for m in maps]

    pct = patches(rz_ct[0])
    praw = patches(rz_raw[0])
    nct = [_rpad(_rpad(patches(rz_ct[k]), PAD2, 1), PAD2, 2) for k in (1, 2)]
    nraw = [_rpad(_rpad(patches(rz_raw[k]), PAD2, 1), PAD2, 2) for k in (1, 2)]

    best_d = None
    best_raw = None
    for img in range(2):
        for dy in range(NSIZE):
            for dx in range(NSIZE):
                dc = pct - nct[img][:, dy:dy + HS, dx:dx + HS]
                d = jnp.sum(dc * dc, axis=0)
                rr = praw - nraw[img][:, dy:dy + HS, dx:dx + HS]
                r = jnp.sum(rr * rr, axis=0)
                if best_d is None:
                    best_d, best_raw = d, r
                else:
                    upd = d < best_d
                    best_d = jnp.where(upd, d, best_d)
                    best_raw = jnp.where(upd, r, best_raw)
    out_ref[0] = jnp.full((8, 128), jnp.sum(best_raw), jnp.float32)


def kernel(pred, I):
    b = pred.shape[0]
    w_op = jnp.asarray(_RESIZE_W)
    partial = pl.pallas_call(
        _fused_body,
        grid=(b,),
        in_specs=[
            pl.BlockSpec((1, C, H, H), lambda i: (i, 0, 0, 0)),
            pl.BlockSpec((1, 2, C, H, H), lambda i: (i, 0, 0, 0, 0)),
            pl.BlockSpec((HS, H), lambda i: (0, 0)),
        ],
        out_specs=pl.BlockSpec((1, 8, 128), lambda i: (i, 0, 0)),
        out_shape=jax.ShapeDtypeStruct((b, 8, 128), jnp.float32),
    )(pred, I, w_op)
    total = jnp.sum(partial[:, 0, 0])
    return total * (0.5 / (b * HS * HS * C1))


# speedup vs baseline: 11.8693x; 1.9302x over previous
import numpy as np
import jax
import jax.numpy as jnp
from jax.experimental import pallas as pl

KSIZE = 3
NSIZE = 7
H = 224
HS = 56
C = 3
C1 = C * KSIZE * KSIZE
PAD2 = NSIZE // 2

_HIGH = jax.lax.Precision.HIGHEST


def _resize_matrix(in_size, out_size):
    scale = out_size / in_size
    inv_scale = 1.0 / scale
    kernel_scale = max(inv_scale, 1.0)
    sample_f = (np.arange(out_size, dtype=np.float32) + 0.5) * inv_scale - 0.5
    x = np.abs(sample_f[None, :]
               - np.arange(in_size, dtype=np.float32)[:, None]) / kernel_scale
    x = x.astype(np.float32)
    w = (((1.5 * x - 2.5) * x * x + 1.0) * (x <= 1.0)
         + ((((-0.5 * x + 2.5) * x - 4.0) * x + 2.0)
            * ((x > 1.0) & (x < 2.0)))).astype(np.float32)
    total = w.sum(axis=0, keepdims=True)
    w = np.where(np.abs(total) > 1000 * np.finfo(np.float32).eps,
                 w / np.where(total != 0, total, 1), 0)
    w = np.where(((sample_f >= -0.5) & (sample_f <= in_size - 0.5))[None, :],
                 w, 0)
    return np.ascontiguousarray(w.T.astype(np.float32))


_RESIZE_W = _resize_matrix(H, HS)


def _rpad(x, p, axis):
    n = x.shape[axis]

    def sl(a, b):
        return tuple(slice(a, b) if d == axis else slice(None)
                     for d in range(x.ndim))

    parts = ([x[sl(t, t + 1)] for t in range(p, 0, -1)]
             + [x]
             + [x[sl(n - 1 - t, n - t)] for t in range(1, p + 1)])
    return jnp.concatenate(parts, axis=axis)


def _fused_body(pred_ref, i_ref, w_ref, out_ref):
    w = w_ref[...]

    def census(x):
        xp = _rpad(_rpad(x, 1, 1), 1, 2)
        acc = jnp.zeros((C, H, H), jnp.float32)
        for i in range(KSIZE):
            for j in range(KSIZE):
                acc = acc + jnp.tanh(xp[:, i:i + H, j:j + H] - x)
        return acc * (1.0 / (KSIZE * KSIZE))

    def resize(m):
        t1 = jax.lax.dot_general(m, w, (((1,), (1,)), ((), ())),
                                 precision=_HIGH)
        return jax.lax.dot_general(t1, w, (((1,), (1,)), ((), ())),
                                   precision=_HIGH)

    def patches(r):
        rp = _rpad(_rpad(r, 1, 1), 1, 2)
        cols = [rp[:, i:i + HS, j:j + HS]
                for i in range(KSIZE) for j in range(KSIZE)]
        return jnp.stack(cols, axis=1).reshape(C1, HS, HS)

    maps = [pred_ref[0], i_ref[0, 0], i_ref[0, 1]]
    rz_ct = [resize(census(m)) for m in maps]
    rz_raw = [resize(m) for m in maps]

    pct = patches(rz_ct[0])
    praw = patches(rz_raw[0])
    nct = [_rpad(_rpad(patches(rz_ct[k]), PAD2, 1), PAD2, 2) for k in (1, 2)]
    nraw = [_rpad(_rpad(patches(rz_raw[k]), PAD2, 1), PAD2, 2) for k in (1, 2)]

    HP = HS + 2 * PAD2
    npc = jnp.concatenate(nct, axis=2)
    npr = jnp.concatenate(nraw, axis=2)
    z6 = jnp.zeros((C1, HS, HP - HS), jnp.float32)
    ppc = jnp.concatenate([pct, z6, pct, z6], axis=2)
    ppr = jnp.concatenate([praw, z6, praw, z6], axis=2)
    z6s = jnp.zeros((HS, HP - HS), jnp.float32)
    pss_c = jnp.sum(pct * pct, axis=0)
    pss_r = jnp.sum(praw * praw, axis=0)
    pssp_c = jnp.concatenate([pss_c, z6s, pss_c, z6s], axis=1)
    pssp_r = jnp.concatenate([pss_r, z6s, pss_r, z6s], axis=1)
    nss_c = jnp.sum(npc * npc, axis=0)
    nss_r = jnp.sum(npr * npr, axis=0)

    best_d = None
    best_raw = None
    for dx in range(NSIZE):
        ppc_dx = jnp.roll(ppc, dx, axis=2) if dx else ppc
        ppr_dx = jnp.roll(ppr, dx, axis=2) if dx else ppr
        def unroll(x):
            return jnp.roll(x, -dx, axis=1) if dx else x

        for dy in range(NSIZE):
            cross_c = jnp.sum(ppc_dx * npc[:, dy:dy + HS, :], axis=0)
            d = unroll(nss_c[dy:dy + HS, :] - 2.0 * cross_c) + pssp_c
            cross_r = jnp.sum(ppr_dx * npr[:, dy:dy + HS, :], axis=0)
            r = unroll(nss_r[dy:dy + HS, :] - 2.0 * cross_r) + pssp_r
            if best_d is None:
                best_d, best_raw = d, r
            else:
                upd = d < best_d
                best_d = jnp.where(upd, d, best_d)
                best_raw = jnp.where(upd, r, best_raw)
    d0, d1 = best_d[:, 0:HS], best_d[:, HP:HP + HS]
    r0, r1 = best_raw[:, 0:HS], best_raw[:, HP:HP + HS]
    final_raw = jnp.where(d1 < d0, r1, r0)
    out_ref[0] = jnp.full((8, 128), jnp.sum(final_raw), jnp.float32)


def kernel(pred, I):
    b = pred.shape[0]
    w_op = jnp.asarray(_RESIZE_W)
    partial = pl.pallas_call(
        _fused_body,
        grid=(b,),
        in_specs=[
            pl.BlockSpec((1, C, H, H), lambda i: (i, 0, 0, 0)),
            pl.BlockSpec((1, 2, C, H, H), lambda i: (i, 0, 0, 0, 0)),
            pl.BlockSpec((HS, H), lambda i: (0, 0)),
        ],
        out_specs=pl.BlockSpec((1, 8, 128), lambda i: (i, 0, 0)),
        out_shape=jax.ShapeDtypeStruct((b, 8, 128), jnp.float32),
    )(pred, I, w_op)
    total = jnp.sum(partial[:, 0, 0])
    return total * (0.5 / (b * HS * HS * C1))
